# baseline (speedup 1.0000x reference)
import jax
import jax.numpy as jnp
from jax import lax
from jax.experimental import pallas as pl
from jax.experimental.pallas import tpu as pltpu

_N = 1024
_H = 28
_W = 28
_NC = 81
_B = 8
_NBLK = _N // _B


def _body(ids_ref, t_ref, p_ref, out_ref):
    i = pl.program_id(0)
    ids = ids_ref[0, 0, :]
    t = t_ref[0]
    pred = p_ref[...]

    cls = lax.broadcasted_iota(jnp.int32, (_B, _NC), 1)
    oh = (ids[:, None] == cls).astype(jnp.float32)
    yp = lax.dot_general(
        pred, oh,
        dimension_numbers=(((3,), (1,)), ((0,), (0,))),
        preferred_element_type=jnp.float32,
    )

    eps = jnp.float32(1e-7)
    p = jnp.clip(yp, eps, 1.0 - eps)
    bce = -(t * jnp.log(p) + (1.0 - t) * jnp.log(1.0 - p))
    valid = (ids > 0).astype(jnp.float32)
    sroi = jnp.sum(bce, axis=(1, 2))
    bsum = jnp.sum(sroi * valid)
    bcnt = jnp.sum(valid)

    @pl.when(i == 0)
    def _init():
        out_ref[0, 0] = 0.0
        out_ref[0, 1] = 0.0

    out_ref[0, 0] += bsum
    out_ref[0, 1] += bcnt

    @pl.when(i == _NBLK - 1)
    def _fini():
        total = out_ref[0, 0]
        cnt = out_ref[0, 1]
        denom = jnp.maximum(cnt, 1.0) * jnp.float32(_H * _W)
        out_ref[0, 0] = jnp.where(cnt > 0, total / denom, jnp.float32(0.0))


@jax.jit
def kernel(true_masks, target_class_ids, pred_masks):
    ids = target_class_ids.reshape(_N).astype(jnp.int32)
    pred4 = pred_masks.reshape(_N, _H, _W, _NC)
    t4 = true_masks.reshape(_NBLK, _B, _H, _W)

    out = pl.pallas_call(
        _body,
        grid=(_NBLK,),
        in_specs=[
            pl.BlockSpec((1, 1, _B), lambda i: (i, 0, 0)),
            pl.BlockSpec((1, _B, _H, _W), lambda i: (i, 0, 0, 0)),
            pl.BlockSpec((_B, _H, _W, _NC), lambda i: (i, 0, 0, 0)),
        ],
        out_specs=pl.BlockSpec(
            (1, 2), lambda i: (0, 0), memory_space=pltpu.SMEM
        ),
        out_shape=jax.ShapeDtypeStruct((1, 2), jnp.float32),
    )(ids.reshape(_NBLK, 1, _B), t4, pred4)
    return out[0, 0]
